# baseline (speedup 1.0000x reference)
import jax
import jax.numpy as jnp
from jax.experimental import pallas as pl
from jax.experimental.pallas import tpu as pltpu

_VOCAB = 130000
_HIDDEN = 128
_VT = 16384
_GRID = (_VOCAB + _VT - 1) // _VT


def _fused_body(ids_ref, embed_ref, w_hbm, b_hbm, out_hbm, x_ref, flag_ref,
                sem):
    n = x_ref.shape[0]

    def _start(i, c):
        pltpu.make_async_copy(
            embed_ref.at[pl.ds(ids_ref[i], 1), :],
            x_ref.at[pl.ds(i, 1), :],
            sem).start()
        return c

    jax.lax.fori_loop(0, n, _start, 0)
    flag_ref[0] = 0

    def _tile(w_vmem, b_vmem, out_vmem):
        @pl.when(flag_ref[0] == 0)
        def _drain():
            def _wait(i, c):
                pltpu.make_async_copy(
                    embed_ref.at[pl.ds(0, 1), :],
                    x_ref.at[pl.ds(i, 1), :],
                    sem).wait()
                return c

            jax.lax.fori_loop(0, n, _wait, 0)
            flag_ref[0] = 1

        acc = jax.lax.dot_general(
            x_ref[...], w_vmem[...], (((1,), (1,)), ((), ())),
            preferred_element_type=jnp.float32,
            precision=jax.lax.Precision.DEFAULT)
        out_vmem[...] = acc + b_vmem[...]

    pltpu.emit_pipeline(
        _tile,
        grid=(_GRID,),
        in_specs=[
            pl.BlockSpec((_VT, _HIDDEN), lambda j: (j, 0)),
            pl.BlockSpec((1, _VT), lambda j: (0, j)),
        ],
        out_specs=[pl.BlockSpec((x_ref.shape[0], _VT), lambda j: (0, j))],
    )(w_hbm, b_hbm, out_hbm)


def kernel(input_ids, embed_w, head_w, head_b):
    B, Q = input_ids.shape
    n = B * Q
    ids = jnp.clip(input_ids.reshape(n).astype(jnp.int32), 0, _VOCAB - 1)

    bias2 = head_b.reshape(1, _VOCAB)
    out = pl.pallas_call(
        _fused_body,
        in_specs=[
            pl.BlockSpec(memory_space=pltpu.SMEM),
            pl.BlockSpec(memory_space=pltpu.MemorySpace.HBM),
            pl.BlockSpec(memory_space=pltpu.MemorySpace.HBM),
            pl.BlockSpec(memory_space=pltpu.MemorySpace.HBM),
        ],
        out_specs=pl.BlockSpec(memory_space=pltpu.MemorySpace.HBM),
        out_shape=jax.ShapeDtypeStruct((n, _VOCAB), jnp.float32),
        scratch_shapes=[pltpu.VMEM((n, _HIDDEN), jnp.float32),
                        pltpu.SMEM((1,), jnp.int32),
                        pltpu.SemaphoreType.DMA],
    )(ids, embed_w, head_w, bias2)
    return out.reshape(B, Q, _VOCAB)


# speedup vs baseline: 1.0230x; 1.0230x over previous
import jax
import jax.numpy as jnp
from jax.experimental import pallas as pl
from jax.experimental.pallas import tpu as pltpu

_VOCAB = 130000
_HIDDEN = 128
_VT = 16384
_GRID = (_VOCAB + _VT - 1) // _VT


def _fused_body(ids_ref, embed_ref, w_ref, b_ref, out_ref, x_ref, sem):
    n = x_ref.shape[0]

    @pl.when(pl.program_id(0) == 0)
    def _gather():
        def _start(i, c):
            pltpu.make_async_copy(
                embed_ref.at[pl.ds(ids_ref[i], 1), :],
                x_ref.at[pl.ds(i, 1), :],
                sem).start()
            return c

        jax.lax.fori_loop(0, n, _start, 0, unroll=16)

        def _wait(i, c):
            pltpu.make_async_copy(
                embed_ref.at[pl.ds(ids_ref[i], 1), :],
                x_ref.at[pl.ds(i, 1), :],
                sem).wait()
            return c

        jax.lax.fori_loop(0, n, _wait, 0, unroll=16)

    acc = jax.lax.dot_general(
        x_ref[...], w_ref[...], (((1,), (1,)), ((), ())),
        preferred_element_type=jnp.float32,
        precision=jax.lax.Precision.DEFAULT)
    out_ref[...] = acc + b_ref[...]


def kernel(input_ids, embed_w, head_w, head_b):
    B, Q = input_ids.shape
    n = B * Q
    ids = jnp.clip(input_ids.reshape(n).astype(jnp.int32), 0, _VOCAB - 1)

    bias2 = head_b.reshape(1, _VOCAB)
    out = pl.pallas_call(
        _fused_body,
        grid=(_GRID,),
        in_specs=[
            pl.BlockSpec(memory_space=pltpu.SMEM),
            pl.BlockSpec(memory_space=pltpu.MemorySpace.HBM),
            pl.BlockSpec((_VT, _HIDDEN), lambda j: (j, 0)),
            pl.BlockSpec((1, _VT), lambda j: (0, j)),
        ],
        out_specs=pl.BlockSpec((n, _VT), lambda j: (0, j)),
        out_shape=jax.ShapeDtypeStruct((n, _VOCAB), jnp.float32),
        scratch_shapes=[pltpu.VMEM((n, _HIDDEN), jnp.float32),
                        pltpu.SemaphoreType.DMA],
        compiler_params=pltpu.CompilerParams(
            dimension_semantics=(pltpu.ARBITRARY,)),
    )(ids, embed_w, head_w, bias2)
    return out.reshape(B, Q, _VOCAB)
